# Initial kernel scaffold; baseline (speedup 1.0000x reference)
import jax
import jax.numpy as jnp
from jax.experimental import pallas as pl


def kernel(indices, table):
    raise NotImplementedError("write your pallas kernel here")


# speedup vs baseline: 2.7190x; 2.7190x over previous
import functools

import jax
import jax.numpy as jnp
from jax import lax
from jax.experimental import pallas as pl
from jax.experimental.pallas import tpu as pltpu
from jax.experimental.pallas import tpu_sc as plsc

_INFO = plsc.get_sparse_core_info()
_NC, _NS, _LANES = _INFO.num_cores, _INFO.num_subcores, _INFO.num_lanes
_NW = _NC * _NS

_CB = 32


def _tree_sum(vals):
    while len(vals) > 1:
        nxt = [vals[i] + vals[i + 1] for i in range(0, len(vals) - 1, 2)]
        if len(vals) % 2:
            nxt.append(vals[-1])
        vals = nxt
    return vals[0]


@functools.lru_cache(maxsize=None)
def _build(B, L, D):
    assert D % _LANES == 0
    assert B % (_NW * _CB) == 0
    bpw = B // _NW
    nchunks = bpw // _CB
    nsub = D // _LANES
    scale = 1.0 / L

    mesh = plsc.VectorSubcoreMesh(core_axis_name="c", subcore_axis_name="s")

    @functools.partial(
        pl.kernel,
        out_type=jax.ShapeDtypeStruct((B, D), jnp.float32),
        mesh=mesh,
        compiler_params=pltpu.CompilerParams(use_tc_tiling_on_sc=False),
        scratch_types=[
            pltpu.VMEM((_CB * L,), jnp.int32),
            pltpu.VMEM((_CB * L, D), jnp.float32),
            pltpu.VMEM((_CB, D), jnp.float32),
            pltpu.SemaphoreType.DMA,
        ],
    )
    def k(idx_hbm, table_hbm, out_hbm, idx_v, rows_v, out_v, sem):
        wid = lax.axis_index("s") * _NC + lax.axis_index("c")
        base = wid * bpw

        def chunk_body(c, carry):
            eb = base + c * _CB
            pltpu.sync_copy(idx_hbm.at[pl.ds(eb * L, _CB * L)], idx_v)
            pltpu.async_copy(table_hbm.at[idx_v], rows_v, sem).wait()

            def elem_body(e, carry2):
                r = e * L
                for s in range(nsub):
                    parts = [
                        rows_v[r + j, pl.ds(s * _LANES, _LANES)]
                        for j in range(L)
                    ]
                    out_v[e, pl.ds(s * _LANES, _LANES)] = (
                        _tree_sum(parts) * scale
                    )
                return carry2

            lax.fori_loop(0, _CB, elem_body, 0)
            pltpu.sync_copy(out_v, out_hbm.at[pl.ds(eb, _CB)])
            return carry

        lax.fori_loop(0, nchunks, chunk_body, 0)

    return k


def kernel(indices, table):
    B, L = indices.shape
    D = table.shape[1]
    return _build(B, L, D)(indices.reshape(B * L), table)
